# baseline (speedup 1.0000x reference)
import functools

import jax
import jax.numpy as jnp
from jax import lax
from jax.experimental import pallas as pl
from jax.experimental.pallas import tpu as pltpu
from jax.experimental.pallas import tpu_sc as plsc

N = 10000
NP = 10240
E = 320000
F_IN = 128
HID = 128
C = 7
CP = 16

NC = 2
NS = 16
NW = NC * NS
EPT = E // NW
K = 80
CH = EPT // K
KC = K
CHC = CH
RPT = NP // NS

_mesh = plsc.VectorSubcoreMesh(core_axis_name="c", subcore_axis_name="s")


def _zero_vmem(ref, rows, width):
    per = width // 16
    zv = jnp.zeros((16,), jnp.float32)

    def body(i, carry):
        r = i // per
        j = i % per
        ref[r, pl.ds(j * 16, 16)] = zv
        return carry

    lax.fori_loop(0, rows * per, body, 0)


@functools.partial(
    pl.kernel,
    mesh=_mesh,
    out_type=jax.ShapeDtypeStruct((NC, NP), jnp.float32),
    scratch_types=[
        pltpu.VMEM((CHC, KC), jnp.int32),
        pltpu.VMEM((KC,), jnp.float32),
        pltpu.VMEM((RPT,), jnp.float32),
        pltpu.VMEM_SHARED((NP,), jnp.float32),
    ],
)
def _sc_count(dst_hbm, out_hbm, dst_v, ones_v, zb_v, acc_sh):
    ci = lax.axis_index("c")
    si = lax.axis_index("s")

    def setz(i, carry):
        zb_v[pl.ds(i * 16, 16)] = jnp.zeros((16,), jnp.float32)
        return carry

    lax.fori_loop(0, RPT // 16, setz, 0)

    def seto(i, carry):
        ones_v[pl.ds(i * 16, 16)] = jnp.ones((16,), jnp.float32)
        return carry

    lax.fori_loop(0, KC // 16, seto, 0)

    pltpu.sync_copy(zb_v, acc_sh.at[pl.ds(si * RPT, RPT)])
    plsc.subcore_barrier()

    pltpu.sync_copy(dst_hbm.at[ci, si], dst_v)

    def body(i, carry):
        pltpu.sync_copy(ones_v, acc_sh.at[dst_v.at[i]], add=True)
        return carry

    lax.fori_loop(0, CHC, body, 0)
    plsc.subcore_barrier()

    pltpu.sync_copy(acc_sh.at[pl.ds(si * RPT, RPT)], zb_v)
    pltpu.sync_copy(zb_v, out_hbm.at[ci, pl.ds(si * RPT, RPT)])


def _make_sc_agg(W):
    @functools.partial(
        pl.kernel,
        mesh=_mesh,
        out_type=jax.ShapeDtypeStruct((NC, NP, W), jnp.float32),
        scratch_types=[
            pltpu.VMEM((EPT,), jnp.int32),
            pltpu.VMEM((CH, K), jnp.int32),
            pltpu.VMEM((K, W), jnp.float32),
            pltpu.VMEM((K, W), jnp.float32),
            pltpu.VMEM_SHARED((NP, W), jnp.float32),
            pltpu.SemaphoreType.DMA,
            pltpu.SemaphoreType.DMA,
            pltpu.SemaphoreType.DMA,
            pltpu.SemaphoreType.DMA,
        ],
    )
    def agg(g_hbm, src_hbm, dst_hbm, out_hbm, src_v, dst_v, ra_v, rb_v,
            acc_sh, gsa, gsb, ssa, ssb):
        ci = lax.axis_index("c")
        si = lax.axis_index("s")

        pltpu.async_copy(src_hbm.at[ci, si], src_v, gsa)
        pltpu.async_copy(dst_hbm.at[ci, si], dst_v, gsb)
        _zero_vmem(ra_v, K, W)
        for kk in range(RPT // K):
            pltpu.async_copy(ra_v, acc_sh.at[pl.ds(si * RPT + kk * K, K)],
                             ssa)
        for kk in range(RPT // K):
            pltpu.make_async_copy(
                ra_v, acc_sh.at[pl.ds(si * RPT + kk * K, K)], ssa).wait()
        pltpu.make_async_copy(src_hbm.at[ci, si], src_v, gsa).wait()
        pltpu.make_async_copy(dst_hbm.at[ci, si], dst_v, gsb).wait()
        plsc.subcore_barrier()

        def g_idx(i):
            return src_v.at[pl.ds(i * K, K)]

        pltpu.async_copy(g_hbm.at[g_idx(0)], ra_v, gsa)

        def body(t, carry):
            pltpu.make_async_copy(g_hbm.at[g_idx(2 * t)], ra_v, gsa).wait()
            pltpu.async_copy(ra_v, acc_sh.at[dst_v.at[2 * t]], ssa, add=True)

            @pl.when(t > 0)
            def _():
                pltpu.make_async_copy(
                    rb_v, acc_sh.at[dst_v.at[2 * t - 1]], ssb).wait()

            pltpu.async_copy(g_hbm.at[g_idx(2 * t + 1)], rb_v, gsb)
            pltpu.make_async_copy(
                ra_v, acc_sh.at[dst_v.at[2 * t]], ssa).wait()
            pltpu.async_copy(g_hbm.at[g_idx(2 * t + 2)], ra_v, gsa)
            pltpu.make_async_copy(g_hbm.at[g_idx(2 * t + 1)], rb_v, gsb).wait()
            pltpu.async_copy(
                rb_v, acc_sh.at[dst_v.at[2 * t + 1]], ssb, add=True)
            return carry

        lax.fori_loop(0, (CH - 1) // 2, body, 0)
        pltpu.make_async_copy(g_hbm.at[g_idx(CH - 1)], ra_v, gsa).wait()
        pltpu.make_async_copy(rb_v, acc_sh.at[dst_v.at[CH - 2]], ssb).wait()
        pltpu.sync_copy(ra_v, acc_sh.at[dst_v.at[CH - 1]], add=True)
        plsc.subcore_barrier()

        bufs = [(ra_v, gsa, ssa), (rb_v, gsb, ssb)]
        nwo = RPT // K
        for kk in range(nwo):
            buf, gs, ss = bufs[kk % 2]
            r0 = si * RPT + kk * K
            if kk >= 2:
                p0 = si * RPT + (kk - 2) * K
                pltpu.make_async_copy(
                    buf, out_hbm.at[ci, pl.ds(p0, K)], ss).wait()
            pltpu.async_copy(acc_sh.at[pl.ds(r0, K)], buf, gs)
            pltpu.make_async_copy(acc_sh.at[pl.ds(r0, K)], buf, gs).wait()
            pltpu.async_copy(buf, out_hbm.at[ci, pl.ds(r0, K)], ss)
        for kk in range(max(nwo - 2, 0), nwo):
            buf, gs, ss = bufs[kk % 2]
            r0 = si * RPT + kk * K
            pltpu.make_async_copy(
                buf, out_hbm.at[ci, pl.ds(r0, K)], ss).wait()

    return agg


_sc_agg_128 = _make_sc_agg(HID)


_BN = 2048
_GRID = NP // _BN


def _tc_a_body(x_ref, w_ref, d0_ref, d1_ref, g_ref, dis_ref):
    deg = 1.0 + d0_ref[...] + d1_ref[...]
    dis = lax.rsqrt(deg)
    dis_ref[...] = dis
    g_ref[...] = jnp.dot(x_ref[...], w_ref[...],
                         preferred_element_type=jnp.float32) * dis


def _tc_a(x, W1, d0, d1):
    return pl.pallas_call(
        _tc_a_body,
        grid=(_GRID,),
        in_specs=[
            pl.BlockSpec((_BN, F_IN), lambda i: (i, 0)),
            pl.BlockSpec((F_IN, HID), lambda i: (0, 0)),
            pl.BlockSpec((_BN, 1), lambda i: (i, 0)),
            pl.BlockSpec((_BN, 1), lambda i: (i, 0)),
        ],
        out_specs=[
            pl.BlockSpec((_BN, HID), lambda i: (i, 0)),
            pl.BlockSpec((_BN, 1), lambda i: (i, 0)),
        ],
        out_shape=[
            jax.ShapeDtypeStruct((NP, HID), jnp.float32),
            jax.ShapeDtypeStruct((NP, 1), jnp.float32),
        ],
    )(x, W1, d0, d1)


def _tc_b_body(acc_ref0, acc_ref1, g1_ref, dis_ref, b1_ref, gh_ref):
    tot = acc_ref0[0] + acc_ref1[0] + g1_ref[...]
    h = jnp.maximum(tot * dis_ref[...] + b1_ref[...], 0.0)
    gh_ref[...] = h * dis_ref[...]


def _tc_b(acc1, g1, dis, b1):
    return pl.pallas_call(
        _tc_b_body,
        grid=(_GRID,),
        in_specs=[
            pl.BlockSpec((1, _BN, HID), lambda i: (0, i, 0)),
            pl.BlockSpec((1, _BN, HID), lambda i: (1, i, 0)),
            pl.BlockSpec((_BN, HID), lambda i: (i, 0)),
            pl.BlockSpec((_BN, 1), lambda i: (i, 0)),
            pl.BlockSpec((1, HID), lambda i: (0, 0)),
        ],
        out_specs=pl.BlockSpec((_BN, HID), lambda i: (i, 0)),
        out_shape=jax.ShapeDtypeStruct((NP, HID), jnp.float32),
    )(acc1, acc1, g1, dis, b1)


def _tc_c_body(acc_ref0, acc_ref1, gh_ref, dis_ref, w2_ref, b2_ref, out_ref):
    tot = (acc_ref0[0] + acc_ref1[0] + gh_ref[...]) * dis_ref[...]
    out_ref[...] = jnp.dot(tot, w2_ref[...],
                           preferred_element_type=jnp.float32) + b2_ref[...]


def _tc_c(acc2, gh, dis, W2p, b2p):
    return pl.pallas_call(
        _tc_c_body,
        grid=(_GRID,),
        in_specs=[
            pl.BlockSpec((1, _BN, HID), lambda i: (0, i, 0)),
            pl.BlockSpec((1, _BN, HID), lambda i: (1, i, 0)),
            pl.BlockSpec((_BN, HID), lambda i: (i, 0)),
            pl.BlockSpec((_BN, 1), lambda i: (i, 0)),
            pl.BlockSpec((HID, CP), lambda i: (0, 0)),
            pl.BlockSpec((1, CP), lambda i: (0, 0)),
        ],
        out_specs=pl.BlockSpec((_BN, CP), lambda i: (i, 0)),
        out_shape=jax.ShapeDtypeStruct((NP, CP), jnp.float32),
    )(acc2, acc2, gh, dis, W2p, b2p)


def kernel(x, edge_index, W1, b1, W2, b2):
    srcp = edge_index[0].reshape(NC, NS, EPT)
    dstp = edge_index[1].reshape(NC, NS, CH, K)
    dst4 = dstp

    W2p = jnp.pad(W2, ((0, 0), (0, CP - C)))
    b2p = jnp.pad(b2, ((0, CP - C),)).reshape(1, CP)
    b1r = b1.reshape(1, HID)

    cnt = _sc_count(dst4)
    d0 = cnt[0].reshape(NP, 1)
    d1 = cnt[1].reshape(NP, 1)

    g1, dis = _tc_a(x, W1, d0, d1)
    acc1 = _sc_agg_128(g1, srcp, dstp)
    gh = _tc_b(acc1, g1, dis, b1r)
    acc2 = _sc_agg_128(gh, srcp, dstp)
    out = _tc_c(acc2, gh, dis, W2p, b2p)
    return out[:N, :C]


# speedup vs baseline: 1.0208x; 1.0208x over previous
import functools

import jax
import jax.numpy as jnp
from jax import lax
from jax.experimental import pallas as pl
from jax.experimental.pallas import tpu as pltpu
from jax.experimental.pallas import tpu_sc as plsc

N = 10000
NP = 10240
E = 320000
F_IN = 128
HID = 128
C = 7
CP = 16

NC = 2
NS = 16
NW = NC * NS
EPT = E // NW
K = 80
CH = EPT // K
KC = K
CHC = CH
RPT = NP // NS

_mesh = plsc.VectorSubcoreMesh(core_axis_name="c", subcore_axis_name="s")


def _zero_vmem(ref, rows, width):
    per = width // 16
    zv = jnp.zeros((16,), jnp.float32)

    def body(i, carry):
        r = i // per
        j = i % per
        ref[r, pl.ds(j * 16, 16)] = zv
        return carry

    lax.fori_loop(0, rows * per, body, 0)


@functools.partial(
    pl.kernel,
    mesh=_mesh,
    out_type=jax.ShapeDtypeStruct((NC, NP), jnp.float32),
    scratch_types=[
        pltpu.VMEM((CHC, KC), jnp.int32),
        pltpu.VMEM((KC,), jnp.float32),
        pltpu.VMEM((RPT,), jnp.float32),
        pltpu.VMEM_SHARED((NP,), jnp.float32),
        pltpu.SemaphoreType.DMA,
    ],
)
def _sc_count(dst_hbm, out_hbm, dst_v, ones_v, zb_v, acc_sh, ss):
    ci = lax.axis_index("c")
    si = lax.axis_index("s")

    def setz(i, carry):
        zb_v[pl.ds(i * 16, 16)] = jnp.zeros((16,), jnp.float32)
        return carry

    lax.fori_loop(0, RPT // 16, setz, 0)

    def seto(i, carry):
        ones_v[pl.ds(i * 16, 16)] = jnp.ones((16,), jnp.float32)
        return carry

    lax.fori_loop(0, KC // 16, seto, 0)

    pltpu.sync_copy(zb_v, acc_sh.at[pl.ds(si * RPT, RPT)])
    plsc.subcore_barrier()

    pltpu.sync_copy(dst_hbm.at[ci, si], dst_v)

    def body(i, carry):
        pltpu.async_copy(ones_v, acc_sh.at[dst_v.at[i]], ss, add=True)

        @pl.when(i >= 4)
        def _():
            pltpu.make_async_copy(ones_v, acc_sh.at[dst_v.at[0]], ss).wait()

        return carry

    lax.fori_loop(0, CHC, body, 0)
    for _ in range(4):
        pltpu.make_async_copy(ones_v, acc_sh.at[dst_v.at[0]], ss).wait()
    plsc.subcore_barrier()

    pltpu.sync_copy(acc_sh.at[pl.ds(si * RPT, RPT)], zb_v)
    pltpu.sync_copy(zb_v, out_hbm.at[ci, pl.ds(si * RPT, RPT)])


def _make_sc_agg(W):
    @functools.partial(
        pl.kernel,
        mesh=_mesh,
        out_type=jax.ShapeDtypeStruct((NC, NP, W), jnp.float32),
        scratch_types=[
            pltpu.VMEM((EPT,), jnp.int32),
            pltpu.VMEM((CH, K), jnp.int32),
            pltpu.VMEM((K, W), jnp.float32),
            pltpu.VMEM((K, W), jnp.float32),
            pltpu.VMEM_SHARED((NP, W), jnp.float32),
            pltpu.SemaphoreType.DMA,
            pltpu.SemaphoreType.DMA,
            pltpu.SemaphoreType.DMA,
            pltpu.SemaphoreType.DMA,
        ],
    )
    def agg(g_hbm, src_hbm, dst_hbm, out_hbm, src_v, dst_v, ra_v, rb_v,
            acc_sh, gsa, gsb, ssa, ssb):
        ci = lax.axis_index("c")
        si = lax.axis_index("s")

        pltpu.async_copy(src_hbm.at[ci, si], src_v, gsa)
        pltpu.async_copy(dst_hbm.at[ci, si], dst_v, gsb)
        _zero_vmem(ra_v, K, W)
        for kk in range(RPT // K):
            pltpu.async_copy(ra_v, acc_sh.at[pl.ds(si * RPT + kk * K, K)],
                             ssa)
        for kk in range(RPT // K):
            pltpu.make_async_copy(
                ra_v, acc_sh.at[pl.ds(si * RPT + kk * K, K)], ssa).wait()
        pltpu.make_async_copy(src_hbm.at[ci, si], src_v, gsa).wait()
        pltpu.make_async_copy(dst_hbm.at[ci, si], dst_v, gsb).wait()
        plsc.subcore_barrier()

        def g_idx(i):
            return src_v.at[pl.ds(i * K, K)]

        pltpu.async_copy(g_hbm.at[g_idx(0)], ra_v, gsa)

        def body(t, carry):
            pltpu.make_async_copy(g_hbm.at[g_idx(2 * t)], ra_v, gsa).wait()
            pltpu.async_copy(ra_v, acc_sh.at[dst_v.at[2 * t]], ssa, add=True)

            @pl.when(t > 0)
            def _():
                pltpu.make_async_copy(
                    rb_v, acc_sh.at[dst_v.at[2 * t - 1]], ssb).wait()

            pltpu.async_copy(g_hbm.at[g_idx(2 * t + 1)], rb_v, gsb)
            pltpu.make_async_copy(
                ra_v, acc_sh.at[dst_v.at[2 * t]], ssa).wait()
            pltpu.async_copy(g_hbm.at[g_idx(2 * t + 2)], ra_v, gsa)
            pltpu.make_async_copy(g_hbm.at[g_idx(2 * t + 1)], rb_v, gsb).wait()
            pltpu.async_copy(
                rb_v, acc_sh.at[dst_v.at[2 * t + 1]], ssb, add=True)
            return carry

        lax.fori_loop(0, (CH - 1) // 2, body, 0)
        pltpu.make_async_copy(g_hbm.at[g_idx(CH - 1)], ra_v, gsa).wait()
        pltpu.make_async_copy(rb_v, acc_sh.at[dst_v.at[CH - 2]], ssb).wait()
        pltpu.sync_copy(ra_v, acc_sh.at[dst_v.at[CH - 1]], add=True)
        plsc.subcore_barrier()

        bufs = [(ra_v, gsa, ssa), (rb_v, gsb, ssb)]
        nwo = RPT // K
        for kk in range(nwo):
            buf, gs, ss = bufs[kk % 2]
            r0 = si * RPT + kk * K
            if kk >= 2:
                p0 = si * RPT + (kk - 2) * K
                pltpu.make_async_copy(
                    buf, out_hbm.at[ci, pl.ds(p0, K)], ss).wait()
            pltpu.async_copy(acc_sh.at[pl.ds(r0, K)], buf, gs)
            pltpu.make_async_copy(acc_sh.at[pl.ds(r0, K)], buf, gs).wait()
            pltpu.async_copy(buf, out_hbm.at[ci, pl.ds(r0, K)], ss)
        for kk in range(max(nwo - 2, 0), nwo):
            buf, gs, ss = bufs[kk % 2]
            r0 = si * RPT + kk * K
            pltpu.make_async_copy(
                buf, out_hbm.at[ci, pl.ds(r0, K)], ss).wait()

    return agg


_sc_agg_128 = _make_sc_agg(HID)


_BN = 2048
_GRID = NP // _BN


def _tc_a_body(x_ref, w_ref, d0_ref, d1_ref, g_ref, dis_ref):
    deg = 1.0 + d0_ref[...] + d1_ref[...]
    dis = lax.rsqrt(deg)
    dis_ref[...] = dis
    g_ref[...] = jnp.dot(x_ref[...], w_ref[...],
                         preferred_element_type=jnp.float32) * dis


def _tc_a(x, W1, d0, d1):
    return pl.pallas_call(
        _tc_a_body,
        grid=(_GRID,),
        in_specs=[
            pl.BlockSpec((_BN, F_IN), lambda i: (i, 0)),
            pl.BlockSpec((F_IN, HID), lambda i: (0, 0)),
            pl.BlockSpec((_BN, 1), lambda i: (i, 0)),
            pl.BlockSpec((_BN, 1), lambda i: (i, 0)),
        ],
        out_specs=[
            pl.BlockSpec((_BN, HID), lambda i: (i, 0)),
            pl.BlockSpec((_BN, 1), lambda i: (i, 0)),
        ],
        out_shape=[
            jax.ShapeDtypeStruct((NP, HID), jnp.float32),
            jax.ShapeDtypeStruct((NP, 1), jnp.float32),
        ],
    )(x, W1, d0, d1)


def _tc_b_body(acc_ref0, acc_ref1, g1_ref, dis_ref, b1_ref, gh_ref):
    tot = acc_ref0[0] + acc_ref1[0] + g1_ref[...]
    h = jnp.maximum(tot * dis_ref[...] + b1_ref[...], 0.0)
    gh_ref[...] = h * dis_ref[...]


def _tc_b(acc1, g1, dis, b1):
    return pl.pallas_call(
        _tc_b_body,
        grid=(_GRID,),
        in_specs=[
            pl.BlockSpec((1, _BN, HID), lambda i: (0, i, 0)),
            pl.BlockSpec((1, _BN, HID), lambda i: (1, i, 0)),
            pl.BlockSpec((_BN, HID), lambda i: (i, 0)),
            pl.BlockSpec((_BN, 1), lambda i: (i, 0)),
            pl.BlockSpec((1, HID), lambda i: (0, 0)),
        ],
        out_specs=pl.BlockSpec((_BN, HID), lambda i: (i, 0)),
        out_shape=jax.ShapeDtypeStruct((NP, HID), jnp.float32),
    )(acc1, acc1, g1, dis, b1)


def _tc_c_body(acc_ref0, acc_ref1, gh_ref, dis_ref, w2_ref, b2_ref, out_ref):
    tot = (acc_ref0[0] + acc_ref1[0] + gh_ref[...]) * dis_ref[...]
    out_ref[...] = jnp.dot(tot, w2_ref[...],
                           preferred_element_type=jnp.float32) + b2_ref[...]


def _tc_c(acc2, gh, dis, W2, b2):
    return pl.pallas_call(
        _tc_c_body,
        grid=(_GRID,),
        in_specs=[
            pl.BlockSpec((1, _BN, HID), lambda i: (0, i, 0)),
            pl.BlockSpec((1, _BN, HID), lambda i: (1, i, 0)),
            pl.BlockSpec((_BN, HID), lambda i: (i, 0)),
            pl.BlockSpec((_BN, 1), lambda i: (i, 0)),
            pl.BlockSpec((HID, C), lambda i: (0, 0)),
            pl.BlockSpec((1, C), lambda i: (0, 0)),
        ],
        out_specs=pl.BlockSpec((_BN, C), lambda i: (i, 0)),
        out_shape=jax.ShapeDtypeStruct((N, C), jnp.float32),
    )(acc2, acc2, gh, dis, W2, b2)


def kernel(x, edge_index, W1, b1, W2, b2):
    srcp = edge_index[0].reshape(NC, NS, EPT)
    dstp = edge_index[1].reshape(NC, NS, CH, K)
    dst4 = dstp

    b1r = b1.reshape(1, HID)
    b2r = b2.reshape(1, C)

    cnt = _sc_count(dst4)
    d0 = cnt[0].reshape(NP, 1)
    d1 = cnt[1].reshape(NP, 1)

    g1, dis = _tc_a(x, W1, d0, d1)
    acc1 = _sc_agg_128(g1, srcp, dstp)
    gh = _tc_b(acc1, g1, dis, b1r)
    acc2 = _sc_agg_128(gh, srcp, dstp)
    return _tc_c(acc2, gh, dis, W2, b2r)
